# baseline (speedup 1.0000x reference)
import functools

import jax
import jax.numpy as jnp
from jax import lax
from jax.experimental import pallas as pl
from jax.experimental.pallas import tpu as pltpu
from jax.experimental.pallas import tpu_sc as plsc

NUM_ROWS = 400
EMB = 32
FEAT = 16
OUT_D = 78
OUT_P = 128
LANES = 16

NC = 2
NS = 16
NW = NC * NS

CHUNK = 320
SOFF = CHUNK + 8


def _sc_embed(x_flat, w_flat, *, num_tokens):
    tok_per_w = num_tokens // NW
    n_chunks = tok_per_w // CHUNK
    assert n_chunks % 2 == 0

    mesh = plsc.VectorSubcoreMesh(core_axis_name="c", subcore_axis_name="s")

    @functools.partial(
        pl.kernel,
        out_type=jax.ShapeDtypeStruct((num_tokens * OUT_P,), jnp.float32),
        mesh=mesh,
        scratch_types=[
            pltpu.VMEM((NUM_ROWS * EMB,), jnp.float32),
            pltpu.VMEM((CHUNK * FEAT,), jnp.float32),
            pltpu.VMEM((CHUNK * FEAT,), jnp.float32),
            pltpu.VMEM((2 * CHUNK + 8,), jnp.float32),
            pltpu.VMEM((CHUNK * OUT_P,), jnp.float32),
            pltpu.VMEM((CHUNK * OUT_P,), jnp.float32),
            pltpu.SemaphoreType.DMA,
            pltpu.SemaphoreType.DMA,
            pltpu.SemaphoreType.DMA,
            pltpu.SemaphoreType.DMA,
        ],
        compiler_params=pltpu.CompilerParams(
            needs_layout_passes=False, disable_bounds_checks=True
        ),
    )
    def k(x_hbm, w_hbm, out_hbm, w_v, xraw0, xraw1, stg_v, out0, out1,
          isem0, isem1, osem0, osem1):
        wid = lax.axis_index("s") * NC + lax.axis_index("c")
        base_tok = wid * tok_per_w
        iota = lax.iota(jnp.int32, LANES)

        pltpu.sync_copy(w_hbm, w_v)

        opmap = jnp.where(iota < 8, iota + (EMB - 1), iota + (2 * EMB - 2))
        smask = (iota == 0) | (iota == 8)
        pmask = ~smask
        smap = jnp.where(iota == 8, SOFF, 0)

        def x_window(ci):
            return x_hbm.at[pl.ds((base_tok + ci * CHUNK) * FEAT, CHUNK * FEAT)]

        def out_window(ci):
            return out_hbm.at[pl.ds((base_tok + ci * CHUNK) * OUT_P, CHUNK * OUT_P)]

        def compute(xraw, out_v):
            @plsc.parallel_loop(0, CHUNK, step=4)
            def tok_body(ti):
                for u in range(4):
                    t = ti + u
                    v = xraw[pl.ds(t * FEAT, LANES)]
                    plsc.store_scatter(out_v, [opmap + t * OUT_P], v, mask=pmask)
                    plsc.store_scatter(stg_v, [smap + t], v, mask=smask)

            @plsc.parallel_loop(0, CHUNK // LANES, step=1, unroll=2)
            def group_body(g):
                tvec = iota + g * LANES
                obase = tvec * OUT_P

                fa = stg_v[pl.ds(g * LANES, LANES)]
                fo = stg_v[pl.ds(SOFF + g * LANES, LANES)]
                ia = jnp.clip(fa, 0.0, float(NUM_ROWS - 1)).astype(jnp.int32)
                io = jnp.clip(fo, 0.0, float(NUM_ROWS - 1)).astype(jnp.int32)
                pa = ia * EMB
                po = io * EMB
                ob_o = obase + (EMB + 7)

                for c in range(EMB):
                    cd = (iota + c) & (EMB - 1)
                    va = plsc.load_gather(w_v, [pa + cd])
                    plsc.store_scatter(out_v, [obase + cd], va)
                    vo = plsc.load_gather(w_v, [po + cd])
                    plsc.store_scatter(out_v, [ob_o + cd], vo)

        def step(ci, j, xraw_cur, xraw_nxt, isem_cur, isem_nxt, out_v, osem):
            @pl.when(ci + 1 < n_chunks)
            def _():
                pltpu.async_copy(x_window(ci + 1), xraw_nxt, isem_nxt)

            pltpu.make_async_copy(x_window(ci), xraw_cur, isem_cur).wait()

            @pl.when(j >= 1)
            def _():
                pltpu.make_async_copy(out_v, out_window(ci - 2), osem).wait()

            compute(xraw_cur, out_v)
            pltpu.async_copy(out_v, out_window(ci), osem)

        pltpu.async_copy(x_window(0), xraw0, isem0)

        def pair_body(j, _):
            step(2 * j, j, xraw0, xraw1, isem0, isem1, out0, osem0)
            step(2 * j + 1, j, xraw1, xraw0, isem1, isem0, out1, osem1)
            return 0

        lax.fori_loop(0, n_chunks // 2, pair_body, 0)
        pltpu.make_async_copy(out0, out_window(n_chunks - 2), osem0).wait()
        pltpu.make_async_copy(out1, out_window(n_chunks - 1), osem1).wait()

    return k(x_flat, w_flat)


def kernel(x, W):
    b, t, f = x.shape
    num_tokens = b * t
    out_flat = _sc_embed(x.reshape(-1), W.reshape(-1), num_tokens=num_tokens)
    return out_flat.reshape(num_tokens, OUT_P)[:, :OUT_D].reshape(b, t, OUT_D)


# speedup vs baseline: 1.0340x; 1.0340x over previous
import functools

import jax
import jax.numpy as jnp
from jax import lax
from jax.experimental import pallas as pl
from jax.experimental.pallas import tpu as pltpu
from jax.experimental.pallas import tpu_sc as plsc

NUM_ROWS = 400
EMB = 32
FEAT = 16
OUT_D = 78
OUT_P = 128
LANES = 16

NC = 2
NS = 16
NW = NC * NS

CHUNK = 320
SOFF = CHUNK + 8


def _sc_embed(x_flat, w_flat, *, num_tokens):
    tok_per_w = num_tokens // NW
    n_chunks = tok_per_w // CHUNK
    assert n_chunks % 2 == 0

    mesh = plsc.VectorSubcoreMesh(core_axis_name="c", subcore_axis_name="s")

    @functools.partial(
        pl.kernel,
        out_type=jax.ShapeDtypeStruct((num_tokens * OUT_P,), jnp.float32),
        mesh=mesh,
        scratch_types=[
            pltpu.VMEM((NUM_ROWS * EMB,), jnp.float32),
            pltpu.VMEM((CHUNK * FEAT,), jnp.float32),
            pltpu.VMEM((CHUNK * FEAT,), jnp.float32),
            pltpu.VMEM((2 * CHUNK + 8,), jnp.float32),
            pltpu.VMEM((CHUNK * OUT_P,), jnp.float32),
            pltpu.VMEM((CHUNK * OUT_P,), jnp.float32),
            pltpu.SemaphoreType.DMA,
            pltpu.SemaphoreType.DMA,
            pltpu.SemaphoreType.DMA,
            pltpu.SemaphoreType.DMA,
        ],
        compiler_params=pltpu.CompilerParams(
            needs_layout_passes=False, disable_bounds_checks=True
        ),
    )
    def k(x_hbm, w_hbm, out_hbm, w_v, xraw0, xraw1, stg_v, out0, out1,
          isem0, isem1, osem0, osem1):
        wid = lax.axis_index("s") * NC + lax.axis_index("c")
        base_tok = wid * tok_per_w
        iota = lax.iota(jnp.int32, LANES)

        pltpu.sync_copy(w_hbm, w_v)

        opmap = jnp.where(iota < 8, iota + (EMB - 1), iota + (2 * EMB - 2))
        smask = (iota == 0) | (iota == 8)
        pmask = ~smask
        smap = jnp.where(iota == 8, SOFF, 0)

        def x_window(ci):
            return x_hbm.at[pl.ds((base_tok + ci * CHUNK) * FEAT, CHUNK * FEAT)]

        def out_window(ci):
            return out_hbm.at[pl.ds((base_tok + ci * CHUNK) * OUT_P, CHUNK * OUT_P)]

        def compute(xraw, out_v):
            @plsc.parallel_loop(0, CHUNK, step=4)
            def tok_body(ti):
                for u in range(4):
                    t = ti + u
                    v = xraw[pl.ds(t * FEAT, LANES)]
                    plsc.store_scatter(out_v, [opmap + t * OUT_P], v, mask=pmask)
                    plsc.store_scatter(stg_v, [smap + t], v, mask=smask)

            @plsc.parallel_loop(0, CHUNK // LANES, step=1)
            def group_body(g):
                tvec = iota + g * LANES
                obase = tvec * OUT_P

                fa = stg_v[pl.ds(g * LANES, LANES)]
                fo = stg_v[pl.ds(SOFF + g * LANES, LANES)]
                ia = jnp.clip(fa, 0.0, float(NUM_ROWS - 1)).astype(jnp.int32)
                io = jnp.clip(fo, 0.0, float(NUM_ROWS - 1)).astype(jnp.int32)
                pa = ia * EMB
                po = io * EMB
                ob_o = obase + (EMB + 7)

                for c in range(EMB):
                    cd = (iota + c) & (EMB - 1)
                    va = plsc.load_gather(w_v, [pa + cd])
                    plsc.store_scatter(out_v, [obase + cd], va)
                    vo = plsc.load_gather(w_v, [po + cd])
                    plsc.store_scatter(out_v, [ob_o + cd], vo)

        def step(ci, j, xraw_cur, xraw_nxt, isem_cur, isem_nxt, out_v, osem):
            @pl.when(ci + 1 < n_chunks)
            def _():
                pltpu.async_copy(x_window(ci + 1), xraw_nxt, isem_nxt)

            pltpu.make_async_copy(x_window(ci), xraw_cur, isem_cur).wait()

            @pl.when(j >= 1)
            def _():
                pltpu.make_async_copy(out_v, out_window(ci - 2), osem).wait()

            compute(xraw_cur, out_v)
            pltpu.async_copy(out_v, out_window(ci), osem)

        pltpu.async_copy(x_window(0), xraw0, isem0)

        def pair_body(j, _):
            step(2 * j, j, xraw0, xraw1, isem0, isem1, out0, osem0)
            step(2 * j + 1, j, xraw1, xraw0, isem1, isem0, out1, osem1)
            return 0

        lax.fori_loop(0, n_chunks // 2, pair_body, 0)
        pltpu.make_async_copy(out0, out_window(n_chunks - 2), osem0).wait()
        pltpu.make_async_copy(out1, out_window(n_chunks - 1), osem1).wait()

    return k(x_flat, w_flat)


def kernel(x, W):
    b, t, f = x.shape
    num_tokens = b * t
    out_flat = _sc_embed(x.reshape(-1), W.reshape(-1), num_tokens=num_tokens)
    return out_flat.reshape(num_tokens, OUT_P)[:, :OUT_D].reshape(b, t, OUT_D)
